# baseline (speedup 1.0000x reference)
import functools

import jax
import jax.numpy as jnp
from jax import lax
from jax.experimental import pallas as pl
from jax.experimental.pallas import tpu as pltpu
from jax.experimental.pallas import tpu_sc as plsc

_NC = 2
_NS = 16
_LANES = 16


def _sc_phase(cg, nf, shock):
    n, h = nf.shape
    k = shock.shape[0]
    kh = k // 2

    mesh = plsc.VectorSubcoreMesh(
        core_axis_name="c", subcore_axis_name="s",
        num_cores=_NC, num_subcores=_NS,
    )

    @functools.partial(
        pl.kernel,
        out_type=[
            jax.ShapeDtypeStruct((k, n), jnp.float32),
            jax.ShapeDtypeStruct((k, h), jnp.float32),
        ],
        mesh=mesh,
        scratch_types=[
            pltpu.VMEM((kh,), jnp.int32),
            pltpu.VMEM((k,), jnp.int32),
            pltpu.VMEM((kh, n), jnp.float32),
            pltpu.VMEM((k, h), jnp.float32),
            pltpu.SemaphoreType.DMA,
        ],
    )
    def sc(cg_hbm, nf_hbm, shock_hbm, wsub_o, nfs_o,
           idxh, idxk, rows, nfsv, sem):
        wid = lax.axis_index("s") * _NC + lax.axis_index("c")

        @pl.when(wid == 0)
        def _():
            pltpu.sync_copy(shock_hbm.at[pl.ds(0, kh)], idxh)
            pltpu.async_copy(cg_hbm.at[idxh], rows, sem).wait()
            pltpu.sync_copy(rows, wsub_o.at[pl.ds(0, kh)])

        @pl.when(wid == 1)
        def _():
            pltpu.sync_copy(shock_hbm.at[pl.ds(kh, kh)], idxh)
            pltpu.async_copy(cg_hbm.at[idxh], rows, sem).wait()
            pltpu.sync_copy(rows, wsub_o.at[pl.ds(kh, kh)])

        @pl.when(wid == 2)
        def _():
            pltpu.sync_copy(shock_hbm, idxk)
            pltpu.async_copy(nf_hbm.at[idxk], nfsv, sem).wait()
            pltpu.sync_copy(nfsv, nfs_o)

    return sc(cg, nf, shock)


def _tc_body(nf_ref, wsub_ref, nfs_ref, f0r_ref, sh_ref,
             w1at_ref, w1bt_ref, uu_ref, vv_ref, cc_ref,
             w2t_ref, b2_ref, w3_ref, b3_ref,
             np_ref, ar_ref, na_ref):
    nf = nf_ref[...]
    nfs = nfs_ref[...]
    dimn = (((1,), (1,)), ((), ()))
    nfb = nf.astype(jnp.bfloat16)
    bmt = lax.dot_general(w1bt_ref[...].astype(jnp.bfloat16), nfb, dimn,
                          preferred_element_type=jnp.float32)
    amt = lax.dot_general(w1at_ref[...], nfs, dimn,
                          preferred_element_type=jnp.float32)
    f0r = f0r_ref[...]
    uu = uu_ref[...]
    vv = vv_ref[...]
    base = bmt + cc_ref[...]
    w2tb = w2t_ref[...].astype(jnp.bfloat16)
    b2c = b2_ref[...]
    w3c = w3_ref[...]
    b3 = b3_ref[0, 0]
    n = nf.shape[0]
    k = nfs.shape[0]

    best = jnp.zeros((1, n), jnp.float32)
    for i in range(k):
        wrow = wsub_ref[i:i + 1, :]
        d = jnp.abs(f0r - nfs[i, 0])
        pre = base + amt[:, i:i + 1] + uu * wrow + vv * d
        h1 = jnp.maximum(pre, 0.0).astype(jnp.bfloat16)
        h2 = jnp.dot(w2tb, h1, preferred_element_type=jnp.float32) + b2c
        h2 = jnp.maximum(h2, 0.0)
        s = jnp.sum(h2 * w3c, axis=0, keepdims=True) + b3
        best = jnp.maximum(best, jax.nn.sigmoid(s) * wrow)

    jcol = lax.broadcasted_iota(jnp.int32, (1, n), 1)
    is_shock = jnp.any(jcol == sh_ref[...], axis=0, keepdims=True)
    p0 = jnp.where(is_shock, 1.0, 0.0)
    a0 = jnp.where(is_shock, 0.0, jnp.inf)
    newp = jnp.maximum(p0, best)
    np_ref[...] = newp
    ar_ref[...] = jnp.where(best > p0, jnp.minimum(a0, 1.0), a0)
    na_ref[...] = jnp.sum((newp > 0.1).astype(jnp.int32)).reshape(1, 1)


def _tc_phase(nf, wsub, nfs, f0r, sh, w1at, w1bt, uu, vv, cc, w2t, b2c, w3c, b3):
    n = nf.shape[0]
    return pl.pallas_call(
        _tc_body,
        out_shape=[
            jax.ShapeDtypeStruct((1, n), jnp.float32),
            jax.ShapeDtypeStruct((1, n), jnp.float32),
            jax.ShapeDtypeStruct((1, 1), jnp.int32),
        ],
    )(nf, wsub, nfs, f0r, sh, w1at, w1bt, uu, vv, cc, w2t, b2c, w3c, b3)


def kernel(causal_graph, node_features, shock_nodes, W1, b1, W2, b2, W3, b3):
    n, h = node_features.shape

    wsub, nfs = _sc_phase(causal_graph, node_features, shock_nodes)

    w1at = W1[:h].T
    w1bt = W1[h:2 * h].T
    uu = W1[2 * h][:, None]
    vv = W1[2 * h + 3][:, None]
    cc = (b1 + W1[2 * h + 1])[:, None]

    newp, arr, naff = _tc_phase(
        node_features, wsub, nfs,
        node_features[:, 0][None, :], shock_nodes[:, None],
        w1at, w1bt, uu, vv, cc,
        W2.T, b2[:, None], W3, b3.reshape(1, 1),
    )
    return newp.reshape(n), arr.reshape(n), naff[0, 0]


# speedup vs baseline: 1.0908x; 1.0908x over previous
import functools

import jax
import jax.numpy as jnp
from jax import lax
from jax.experimental import pallas as pl
from jax.experimental.pallas import tpu as pltpu
from jax.experimental.pallas import tpu_sc as plsc

_NC = 2
_NS = 16
_LANES = 16


def _sc_phase(cg, nf, shock):
    n, h = nf.shape
    k = shock.shape[0]
    kh = k // 2

    mesh = plsc.VectorSubcoreMesh(
        core_axis_name="c", subcore_axis_name="s",
        num_cores=_NC, num_subcores=_NS,
    )

    @functools.partial(
        pl.kernel,
        out_type=[
            jax.ShapeDtypeStruct((k, n), jnp.float32),
            jax.ShapeDtypeStruct((k, h), jnp.float32),
        ],
        mesh=mesh,
        scratch_types=[
            pltpu.VMEM((kh,), jnp.int32),
            pltpu.VMEM((k,), jnp.int32),
            pltpu.VMEM((kh, n), jnp.float32),
            pltpu.VMEM((k, h), jnp.float32),
            pltpu.SemaphoreType.DMA,
        ],
    )
    def sc(cg_hbm, nf_hbm, shock_hbm, wsub_o, nfs_o,
           idxh, idxk, rows, nfsv, sem):
        wid = lax.axis_index("s") * _NC + lax.axis_index("c")

        @pl.when(wid == 0)
        def _():
            pltpu.sync_copy(shock_hbm.at[pl.ds(0, kh)], idxh)
            pltpu.async_copy(cg_hbm.at[idxh], rows, sem).wait()
            pltpu.sync_copy(rows, wsub_o.at[pl.ds(0, kh)])

        @pl.when(wid == 1)
        def _():
            pltpu.sync_copy(shock_hbm.at[pl.ds(kh, kh)], idxh)
            pltpu.async_copy(cg_hbm.at[idxh], rows, sem).wait()
            pltpu.sync_copy(rows, wsub_o.at[pl.ds(kh, kh)])

        @pl.when(wid == 2)
        def _():
            pltpu.sync_copy(shock_hbm, idxk)
            pltpu.async_copy(nf_hbm.at[idxk], nfsv, sem).wait()
            pltpu.sync_copy(nfsv, nfs_o)

    return sc(cg, nf, shock)


def _tc_body(nf_ref, wsub_ref, nfs_ref, f0r_ref, sh_ref,
             w1at_ref, w1bt_ref, uu_ref, vv_ref, cc_ref,
             w2t_ref, b2_ref, w3_ref, b3_ref,
             np_ref, ar_ref, na_ref):
    nf = nf_ref[...]
    nfs = nfs_ref[...]
    dimn = (((1,), (1,)), ((), ()))
    nfb = nf.astype(jnp.bfloat16)
    bmt = lax.dot_general(w1bt_ref[...].astype(jnp.bfloat16), nfb, dimn,
                          preferred_element_type=jnp.float32)
    amt = lax.dot_general(w1at_ref[...], nfs, dimn,
                          preferred_element_type=jnp.float32)
    f0r = f0r_ref[...]
    uu = uu_ref[...]
    vv = vv_ref[...]
    base = bmt + cc_ref[...]
    w2tb = w2t_ref[...].astype(jnp.bfloat16)
    b2c = b2_ref[...]
    w3c = w3_ref[...]
    b3 = b3_ref[0, 0]
    n = nf.shape[0]
    k = nfs.shape[0]

    best = jnp.zeros((1, n), jnp.float32)
    ones = jnp.ones((1, n), jnp.float32)
    for i in range(k):
        wrow = wsub_ref[i:i + 1, :]
        d = jnp.abs(f0r - nfs[i, 0])
        ext = jnp.concatenate([wrow, d, ones], axis=0)
        small = jnp.concatenate([uu, vv, amt[:, i:i + 1]], axis=1)
        pre = base + jnp.dot(small, ext, preferred_element_type=jnp.float32)
        h1 = jnp.maximum(pre, 0.0).astype(jnp.bfloat16)
        h2 = jnp.dot(w2tb, h1, preferred_element_type=jnp.float32) + b2c
        h2 = jnp.maximum(h2, 0.0)
        s = jnp.sum(h2 * w3c, axis=0, keepdims=True) + b3
        best = jnp.maximum(best, jax.nn.sigmoid(s) * wrow)

    jcol = lax.broadcasted_iota(jnp.int32, (1, n), 1)
    is_shock = jnp.any(jcol == sh_ref[...], axis=0, keepdims=True)
    p0 = jnp.where(is_shock, 1.0, 0.0)
    a0 = jnp.where(is_shock, 0.0, jnp.inf)
    newp = jnp.maximum(p0, best)
    np_ref[...] = newp
    ar_ref[...] = jnp.where(best > p0, jnp.minimum(a0, 1.0), a0)
    na_ref[...] = jnp.sum((newp > 0.1).astype(jnp.int32)).reshape(1, 1)


def _tc_phase(nf, wsub, nfs, f0r, sh, w1at, w1bt, uu, vv, cc, w2t, b2c, w3c, b3):
    n = nf.shape[0]
    return pl.pallas_call(
        _tc_body,
        out_shape=[
            jax.ShapeDtypeStruct((1, n), jnp.float32),
            jax.ShapeDtypeStruct((1, n), jnp.float32),
            jax.ShapeDtypeStruct((1, 1), jnp.int32),
        ],
    )(nf, wsub, nfs, f0r, sh, w1at, w1bt, uu, vv, cc, w2t, b2c, w3c, b3)


def kernel(causal_graph, node_features, shock_nodes, W1, b1, W2, b2, W3, b3):
    n, h = node_features.shape

    wsub, nfs = _sc_phase(causal_graph, node_features, shock_nodes)

    w1at = W1[:h].T
    w1bt = W1[h:2 * h].T
    uu = W1[2 * h][:, None]
    vv = W1[2 * h + 3][:, None]
    cc = (b1 + W1[2 * h + 1])[:, None]

    newp, arr, naff = _tc_phase(
        node_features, wsub, nfs,
        node_features[:, 0][None, :], shock_nodes[:, None],
        w1at, w1bt, uu, vv, cc,
        W2.T, b2[:, None], W3, b3.reshape(1, 1),
    )
    return newp.reshape(n), arr.reshape(n), naff[0, 0]
